# baseline (speedup 1.0000x reference)
import dataclasses
import functools

import jax
import jax.numpy as jnp
from jax import lax
from jax.experimental import pallas as pl
from jax.experimental.pallas import tpu as pltpu
from jax.experimental.pallas import tpu_sc as plsc

NC = 2
NS = 16
NW = NC * NS
K = 128
CH = 16
ZCH = 64


def _mesh():
    return plsc.VectorSubcoreMesh(core_axis_name="c", subcore_axis_name="s")


def _no_layout_cp():
    cp = pltpu.CompilerParams()
    if "needs_layout_passes" in pltpu.CompilerParams.__dataclass_fields__:
        cp = dataclasses.replace(cp, needs_layout_passes=False)
    return cp


def _deg_pass(didx, np_pad):
    nblk = didx.shape[0] // NW
    ones_host = jnp.ones((K, 128), jnp.float32)
    zeros_host = jnp.zeros((ZCH, 128), jnp.float32)

    @functools.partial(
        pl.kernel,
        out_type=jax.ShapeDtypeStruct((NC, np_pad, 128), jnp.float32),
        mesh=_mesh(),
        scratch_types=[
            pltpu.VMEM((nblk, K), jnp.int32),
            pltpu.VMEM((K, 128), jnp.float32),
            pltpu.VMEM((ZCH, 128), jnp.float32),
            pltpu.VMEM_SHARED((np_pad, 128), jnp.float32),
        ],
    )
    def kern(didx_hbm, ones_hbm, zeros_hbm, out_hbm, dall, ones_v, zbuf, acc):
        cid = lax.axis_index("c")
        sid = lax.axis_index("s")
        wid = sid * NC + cid
        pltpu.sync_copy(ones_hbm, ones_v)
        pltpu.sync_copy(zeros_hbm, zbuf)
        pltpu.sync_copy(didx_hbm.at[pl.ds(wid * nblk, nblk)], dall)

        @pl.loop(0, np_pad // (ZCH * NS))
        def _(i):
            pltpu.sync_copy(zbuf, acc.at[pl.ds((i * NS + sid) * ZCH, ZCH)])

        plsc.subcore_barrier()

        @pl.loop(0, nblk)
        def _(b):
            pltpu.sync_copy(ones_v, acc.at[dall.at[b]], add=True)

        plsc.subcore_barrier()
        rps = np_pad // NS
        pltpu.sync_copy(acc.at[pl.ds(sid * rps, rps)],
                        out_hbm.at[cid, pl.ds(sid * rps, rps)])

    return kern(didx, ones_host, zeros_host)


def _filter_pass(gidx, didx, aidx, n, np_pad):
    nblk = gidx.shape[0] // NW
    ept = nblk * K
    ep = NW * ept
    a = aidx.shape[0]
    zflags_host = jnp.zeros((np_pad,), jnp.int32)
    junkd_host = jnp.full((ept,), n, jnp.int32)
    junkg_host = jnp.zeros((ept,), jnp.int32)

    @functools.partial(
        pl.kernel,
        out_type=(
            jax.ShapeDtypeStruct((ep,), jnp.int32),
            jax.ShapeDtypeStruct((ep,), jnp.int32),
            jax.ShapeDtypeStruct((NW, 128), jnp.int32),
        ),
        mesh=_mesh(),
        scratch_types=[
            pltpu.VMEM((nblk, K), jnp.int32),
            pltpu.VMEM((nblk, K), jnp.int32),
            pltpu.VMEM((a,), jnp.int32),
            pltpu.VMEM((np_pad,), jnp.int32),
            pltpu.VMEM((ept,), jnp.int32),
            pltpu.VMEM((ept,), jnp.int32),
            pltpu.VMEM((128,), jnp.int32),
        ],
        compiler_params=_no_layout_cp(),
    )
    def kern(gidx_hbm, didx_hbm, aidx_hbm, zflags_hbm, junkd_hbm, junkg_hbm,
             gout_hbm, dout_hbm, counts_hbm,
             gall, dall, abuf, flag_v, gcomp, dcomp, cbuf):
        cid = lax.axis_index("c")
        sid = lax.axis_index("s")
        wid = sid * NC + cid
        pltpu.sync_copy(gidx_hbm.at[pl.ds(wid * nblk, nblk)], gall)
        pltpu.sync_copy(didx_hbm.at[pl.ds(wid * nblk, nblk)], dall)
        pltpu.sync_copy(zflags_hbm, flag_v)
        pltpu.sync_copy(aidx_hbm, abuf)
        pltpu.sync_copy(junkd_hbm, dcomp)
        pltpu.sync_copy(junkg_hbm, gcomp)
        ones16 = jnp.ones((16,), jnp.int32)

        @pl.loop(0, a // 16)
        def _(c):
            plsc.store_scatter(flag_v, [abuf[pl.ds(c * 16, 16)]], ones16)

        def blk_body(b, cnt):
            def chunk(c, cnt):
                dv = dall[b, pl.ds(c * 16, 16)]
                gv = gall[b, pl.ds(c * 16, 16)]
                fl = plsc.load_gather(flag_v, [dv])
                mask = fl > 0
                plsc.store_compressed(dcomp.at[pl.ds(cnt, 16)], dv, mask=mask)
                plsc.store_compressed(gcomp.at[pl.ds(cnt, 16)], gv, mask=mask)
                pc = plsc.all_reduce_population_count(mask)
                return cnt + lax.reduce_max(pc, axes=(0,))

            return lax.fori_loop(0, K // 16, chunk, cnt)

        cnt = lax.fori_loop(0, nblk, blk_body, jnp.int32(0))
        base = wid * ept
        pltpu.sync_copy(dcomp, dout_hbm.at[pl.ds(base, ept)])
        pltpu.sync_copy(gcomp, gout_hbm.at[pl.ds(base, ept)])
        cvec = jnp.full((16,), cnt, jnp.int32)

        @pl.loop(0, 8)
        def _(j):
            cbuf[pl.ds(j * 16, 16)] = cvec

        pltpu.sync_copy(cbuf, counts_hbm.at[wid])

    return kern(gidx, didx, aidx, zflags_host, junkd_host, junkg_host)


def _edge_pass(table, gidx, didx, np_pad, counts=None):
    h = table.shape[1]
    nblk = gidx.shape[0] // NW
    nch = nblk // CH
    zeros_host = jnp.zeros((ZCH, h), jnp.float32)
    dyn = counts is not None

    scratch = [
        pltpu.VMEM((CH, K), jnp.int32),
        pltpu.VMEM((CH, K), jnp.int32),
        pltpu.VMEM((K, h), jnp.float32),
        pltpu.VMEM((K, h), jnp.float32),
        pltpu.VMEM((ZCH, h), jnp.float32),
        pltpu.VMEM_SHARED((np_pad, h), jnp.float32),
        pltpu.SemaphoreType.DMA,
        pltpu.SemaphoreType.DMA,
    ]
    if dyn:
        scratch.append(pltpu.VMEM((128,), jnp.int32))

    @functools.partial(
        pl.kernel,
        out_type=jax.ShapeDtypeStruct((NC, np_pad, h), jnp.float32),
        mesh=_mesh(),
        scratch_types=scratch,
        compiler_params=_no_layout_cp() if dyn else None,
    )
    def kern(table_hbm, gidx_hbm, didx_hbm, zeros_hbm, *rest):
        if dyn:
            counts_hbm, out_hbm, gall, dall, rows0, rows1, zbuf, acc, \
                sem0, sem1, cbuf = rest
        else:
            out_hbm, gall, dall, rows0, rows1, zbuf, acc, sem0, sem1 = rest
        cid = lax.axis_index("c")
        sid = lax.axis_index("s")
        wid = sid * NC + cid
        pltpu.sync_copy(zeros_hbm, zbuf)

        @pl.loop(0, np_pad // (ZCH * NS))
        def _(i):
            pltpu.sync_copy(zbuf, acc.at[pl.ds((i * NS + sid) * ZCH, ZCH)])

        plsc.subcore_barrier()

        def gather_start(b, rows, sem):
            pltpu.async_copy(table_hbm.at[gall.at[b]], rows, sem)

        def gather_wait(b, rows, sem):
            pltpu.make_async_copy(table_hbm.at[gall.at[b]], rows, sem).wait()

        def scat(b, rows):
            pltpu.sync_copy(rows, acc.at[dall.at[b]], add=True)

        def chunk(ci):
            row0 = wid * nblk + ci * CH
            pltpu.sync_copy(gidx_hbm.at[pl.ds(row0, CH)], gall)
            pltpu.sync_copy(didx_hbm.at[pl.ds(row0, CH)], dall)
            gather_start(0, rows0, sem0)

            @pl.loop(0, CH - 2, step=2)
            def _(b):
                gather_wait(b, rows0, sem0)
                gather_start(b + 1, rows1, sem1)
                scat(b, rows0)
                gather_wait(b + 1, rows1, sem1)
                gather_start(b + 2, rows0, sem0)
                scat(b + 1, rows1)

            gather_wait(CH - 2, rows0, sem0)
            gather_start(CH - 1, rows1, sem1)
            scat(CH - 2, rows0)
            gather_wait(CH - 1, rows1, sem1)
            scat(CH - 1, rows1)

        if dyn:
            pltpu.sync_copy(counts_hbm.at[wid], cbuf)
            cnt = lax.reduce_max(cbuf[pl.ds(0, 16)], axes=(0,))
            ncd = (cnt + (CH * K - 1)) // (CH * K)
            lax.fori_loop(0, ncd, lambda ci, c: (chunk(ci), c)[1], 0)
        else:
            pl.loop(0, nch)(chunk)

        plsc.subcore_barrier()
        rps = np_pad // NS
        pltpu.sync_copy(acc.at[pl.ds(sid * rps, rps)],
                        out_hbm.at[cid, pl.ds(sid * rps, rps)])

    if dyn:
        return kern(table, gidx, didx, zeros_host, counts)
    return kern(table, gidx, didx, zeros_host)


def _aspect_gather(hmat, aidx):
    a = aidx.shape[0]
    h = hmat.shape[1]
    apt = a // NW

    @functools.partial(
        pl.kernel,
        out_type=jax.ShapeDtypeStruct((a, h), jnp.float32),
        mesh=_mesh(),
        scratch_types=[
            pltpu.VMEM((apt,), jnp.int32),
            pltpu.VMEM((apt, h), jnp.float32),
            pltpu.SemaphoreType.DMA,
        ],
    )
    def kern(h_hbm, aidx_hbm, out_hbm, ibuf, rows, sem):
        cid = lax.axis_index("c")
        sid = lax.axis_index("s")
        wid = sid * NC + cid
        pltpu.sync_copy(aidx_hbm.at[pl.ds(wid * apt, apt)], ibuf)
        pltpu.async_copy(h_hbm.at[ibuf], rows, sem).wait()
        pltpu.sync_copy(rows, out_hbm.at[pl.ds(wid * apt, apt)])

    return kern(hmat, aidx)


def _dinv_kernel(degs):
    np_pad = degs.shape[1]

    def body(d_ref, o_ref):
        deg = d_ref[0, :, 0:1] + d_ref[1, :, 0:1]
        o_ref[...] = lax.rsqrt(jnp.maximum(deg, 1.0))

    return pl.pallas_call(
        body,
        out_shape=jax.ShapeDtypeStruct((np_pad, 1), jnp.float32),
    )(degs)


def _typed_transform(x, w, dinv, bn=2048):
    np_pad, d = x.shape
    t, _, h = w.shape

    def body(x_ref, w_ref, dv_ref, o_ref):
        o_ref[0] = dv_ref[...] * jnp.dot(
            x_ref[...], w_ref[0], preferred_element_type=jnp.float32)

    return pl.pallas_call(
        body,
        grid=(t, np_pad // bn),
        in_specs=[
            pl.BlockSpec((bn, d), lambda ti, i: (i, 0)),
            pl.BlockSpec((1, d, h), lambda ti, i: (ti, 0, 0)),
            pl.BlockSpec((bn, 1), lambda ti, i: (i, 0)),
        ],
        out_specs=pl.BlockSpec((1, bn, h), lambda ti, i: (ti, i, 0)),
        out_shape=jax.ShapeDtypeStruct((t, np_pad, h), jnp.float32),
    )(x, w, dinv)


def _root_matmul(x, w_root, b, bn=2048):
    np_pad, d = x.shape
    h = w_root.shape[1]

    def body(x_ref, w_ref, b_ref, o_ref):
        o_ref[...] = jnp.dot(
            x_ref[...], w_ref[...], preferred_element_type=jnp.float32
        ) + b_ref[...]

    return pl.pallas_call(
        body,
        grid=(np_pad // bn,),
        in_specs=[
            pl.BlockSpec((bn, d), lambda i: (i, 0)),
            pl.BlockSpec((d, h), lambda i: (0, 0)),
            pl.BlockSpec((1, h), lambda i: (0, 0)),
        ],
        out_specs=pl.BlockSpec((bn, h), lambda i: (i, 0)),
        out_shape=jax.ShapeDtypeStruct((np_pad, h), jnp.float32),
    )(x, w_root, b)


def _combine(acc, dinv, root, relu, bn=2048):
    np_pad, h = root.shape

    def body(a_ref, dv_ref, r_ref, o_ref):
        s = (a_ref[0] + a_ref[1]) * dv_ref[...] + r_ref[...]
        if relu:
            s = jnp.maximum(s, 0.0)
        o_ref[...] = s

    return pl.pallas_call(
        body,
        grid=(np_pad // bn,),
        in_specs=[
            pl.BlockSpec((2, bn, h), lambda i: (0, i, 0)),
            pl.BlockSpec((bn, 1), lambda i: (i, 0)),
            pl.BlockSpec((bn, h), lambda i: (i, 0)),
        ],
        out_specs=pl.BlockSpec((bn, h), lambda i: (i, 0)),
        out_shape=jax.ShapeDtypeStruct((np_pad, h), jnp.float32),
    )(acc, dinv, root)


def _classifier(asp, wc, bc):
    a, h = asp.shape
    c = wc.shape[1]

    def body(x_ref, w_ref, b_ref, o_ref):
        o_ref[...] = jnp.dot(
            x_ref[...], w_ref[...], preferred_element_type=jnp.float32
        ) + b_ref[...]

    return pl.pallas_call(
        body,
        out_shape=jax.ShapeDtypeStruct((a, c), jnp.float32),
    )(asp, wc, bc.reshape(1, c))


def kernel(features, edge_index, aspect_indices, edge_types,
           W1, W1_root, b1, W2, W2_root, b2, Wc, bc):
    n, d = features.shape
    e = edge_index.shape[1]
    t = W1.shape[0]
    h = W1.shape[2]

    np_pad = -(-(n + 1) // (ZCH * NS)) * (ZCH * NS)
    e_pad = -(-e // (CH * K * NW)) * (CH * K * NW)

    src = edge_index[0].astype(jnp.int32)
    dst = edge_index[1].astype(jnp.int32)
    et = edge_types.astype(jnp.int32)
    pad = e_pad - e
    gidx = jnp.concatenate([et * np_pad + src,
                            jnp.zeros((pad,), jnp.int32)]).reshape(-1, K)
    didx = jnp.concatenate([dst, jnp.full((pad,), n, jnp.int32)]).reshape(-1, K)
    aidx = aspect_indices.astype(jnp.int32)

    x = jnp.pad(features, ((0, np_pad - n), (0, 0)))

    degs = _deg_pass(didx, np_pad)
    dinv = _dinv_kernel(degs)
    gidx2, didx2, counts2 = _filter_pass(gidx, didx, aidx, n, np_pad)
    gidx2 = gidx2.reshape(-1, K)
    didx2 = didx2.reshape(-1, K)

    xts1 = _typed_transform(x, W1, dinv).reshape(t * np_pad, h)
    acc1 = _edge_pass(xts1, gidx, didx, np_pad)
    root1 = _root_matmul(x, W1_root, b1.reshape(1, h))
    h1 = _combine(acc1, dinv, root1, relu=True)

    xts2 = _typed_transform(h1, W2, dinv).reshape(t * np_pad, h)
    acc2 = _edge_pass(xts2, gidx2, didx2, np_pad, counts2)
    root2 = _root_matmul(h1, W2_root, b2.reshape(1, h))
    h2 = _combine(acc2, dinv, root2, relu=False)

    asp = _aspect_gather(h2, aidx)
    return _classifier(asp, Wc, bc)


# speedup vs baseline: 2.4569x; 2.4569x over previous
import dataclasses
import functools

import jax
import jax.numpy as jnp
from jax import lax
from jax.experimental import pallas as pl
from jax.experimental.pallas import tpu as pltpu
from jax.experimental.pallas import tpu_sc as plsc

NC = 2
NS = 16
NW = NC * NS
K = 128
CH = 16
ZCH = 64


def _mesh():
    return plsc.VectorSubcoreMesh(core_axis_name="c", subcore_axis_name="s")


def _no_layout_cp():
    cp = pltpu.CompilerParams()
    if "needs_layout_passes" in pltpu.CompilerParams.__dataclass_fields__:
        cp = dataclasses.replace(cp, needs_layout_passes=False)
    return cp


def _deg_pass(didx, np_pad):
    nblk = didx.shape[0] // (K * NW)
    ones_host = jnp.ones((K, 128), jnp.float32)
    zeros_host = jnp.zeros((ZCH, 128), jnp.float32)

    @functools.partial(
        pl.kernel,
        out_type=jax.ShapeDtypeStruct((NC, np_pad, 128), jnp.float32),
        mesh=_mesh(),
        scratch_types=[
            pltpu.VMEM((K,), jnp.int32),
            pltpu.VMEM((K, 128), jnp.float32),
            pltpu.VMEM((ZCH, 128), jnp.float32),
            pltpu.VMEM_SHARED((np_pad, 128), jnp.float32),
        ],
    )
    def kern(didx_hbm, ones_hbm, zeros_hbm, out_hbm, dbuf, ones_v, zbuf, acc):
        cid = lax.axis_index("c")
        sid = lax.axis_index("s")
        wid = sid * NC + cid
        pltpu.sync_copy(ones_hbm, ones_v)
        pltpu.sync_copy(zeros_hbm, zbuf)

        @pl.loop(0, np_pad // (ZCH * NS))
        def _(i):
            pltpu.sync_copy(zbuf, acc.at[pl.ds((i * NS + sid) * ZCH, ZCH)])

        plsc.subcore_barrier()
        base = wid * nblk * K

        @pl.loop(0, nblk)
        def _(b):
            pltpu.sync_copy(didx_hbm.at[pl.ds(base + b * K, K)], dbuf)
            pltpu.sync_copy(ones_v, acc.at[dbuf], add=True)

        plsc.subcore_barrier()
        rps = np_pad // NS
        pltpu.sync_copy(acc.at[pl.ds(sid * rps, rps)],
                        out_hbm.at[cid, pl.ds(sid * rps, rps)])

    return kern(didx, ones_host, zeros_host)


def _filter_pass(gidx, didx, aidx, n, np_pad):
    ep = gidx.shape[0]
    nblk = ep // (K * NW)
    ept = nblk * K
    a = aidx.shape[0]
    zflags_host = jnp.zeros((np_pad,), jnp.int32)
    junkd_host = jnp.full((ept,), n, jnp.int32)
    junkg_host = jnp.zeros((ept,), jnp.int32)

    @functools.partial(
        pl.kernel,
        out_type=(
            jax.ShapeDtypeStruct((ep,), jnp.int32),
            jax.ShapeDtypeStruct((ep,), jnp.int32),
            jax.ShapeDtypeStruct((NW, 128), jnp.int32),
        ),
        mesh=_mesh(),
        scratch_types=[
            pltpu.VMEM((K,), jnp.int32),
            pltpu.VMEM((K,), jnp.int32),
            pltpu.VMEM((a,), jnp.int32),
            pltpu.VMEM((np_pad,), jnp.int32),
            pltpu.VMEM((ept,), jnp.int32),
            pltpu.VMEM((ept,), jnp.int32),
            pltpu.VMEM((128,), jnp.int32),
        ],
        compiler_params=_no_layout_cp(),
    )
    def kern(gidx_hbm, didx_hbm, aidx_hbm, zflags_hbm, junkd_hbm, junkg_hbm,
             gout_hbm, dout_hbm, counts_hbm,
             gbuf, dbuf, abuf, flag_v, gcomp, dcomp, cbuf):
        cid = lax.axis_index("c")
        sid = lax.axis_index("s")
        wid = sid * NC + cid
        pltpu.sync_copy(zflags_hbm, flag_v)
        pltpu.sync_copy(aidx_hbm, abuf)
        pltpu.sync_copy(junkd_hbm, dcomp)
        pltpu.sync_copy(junkg_hbm, gcomp)
        ones16 = jnp.ones((16,), jnp.int32)

        @pl.loop(0, a // 16)
        def _(c):
            plsc.store_scatter(flag_v, [abuf[pl.ds(c * 16, 16)]], ones16)

        base = wid * ept

        def blk_body(b, cnt):
            pltpu.sync_copy(gidx_hbm.at[pl.ds(base + b * K, K)], gbuf)
            pltpu.sync_copy(didx_hbm.at[pl.ds(base + b * K, K)], dbuf)

            def chunk(c, cnt):
                dv = dbuf[pl.ds(c * 16, 16)]
                gv = gbuf[pl.ds(c * 16, 16)]
                fl = plsc.load_gather(flag_v, [dv])
                mask = fl > 0
                plsc.store_compressed(dcomp.at[pl.ds(cnt, 16)], dv, mask=mask)
                plsc.store_compressed(gcomp.at[pl.ds(cnt, 16)], gv, mask=mask)
                pc = plsc.all_reduce_population_count(mask)
                return cnt + lax.reduce_max(pc, axes=(0,))

            return lax.fori_loop(0, K // 16, chunk, cnt)

        cnt = lax.fori_loop(0, nblk, blk_body, jnp.int32(0))
        pltpu.sync_copy(dcomp, dout_hbm.at[pl.ds(base, ept)])
        pltpu.sync_copy(gcomp, gout_hbm.at[pl.ds(base, ept)])
        cvec = jnp.full((16,), cnt, jnp.int32)

        @pl.loop(0, 8)
        def _(j):
            cbuf[pl.ds(j * 16, 16)] = cvec

        pltpu.sync_copy(cbuf, counts_hbm.at[wid])

    return kern(gidx, didx, aidx, zflags_host, junkd_host, junkg_host)


def _edge_pass(table, gidx, didx, np_pad, counts=None):
    h = table.shape[1]
    nblk = gidx.shape[0] // (K * NW)
    zeros_host = jnp.zeros((ZCH, h), jnp.float32)
    dyn = counts is not None

    scratch = [
        pltpu.VMEM((K,), jnp.int32),
        pltpu.VMEM((K,), jnp.int32),
        pltpu.VMEM((K, h), jnp.float32),
        pltpu.VMEM((ZCH, h), jnp.float32),
        pltpu.VMEM_SHARED((np_pad, h), jnp.float32),
        pltpu.SemaphoreType.DMA,
    ]
    if dyn:
        scratch.append(pltpu.VMEM((128,), jnp.int32))

    @functools.partial(
        pl.kernel,
        out_type=jax.ShapeDtypeStruct((NC, np_pad, h), jnp.float32),
        mesh=_mesh(),
        scratch_types=scratch,
        compiler_params=_no_layout_cp() if dyn else None,
    )
    def kern(table_hbm, gidx_hbm, didx_hbm, zeros_hbm, *rest):
        if dyn:
            counts_hbm, out_hbm, gbuf, dbuf, rows, zbuf, acc, sem, cbuf = rest
        else:
            out_hbm, gbuf, dbuf, rows, zbuf, acc, sem = rest
        cid = lax.axis_index("c")
        sid = lax.axis_index("s")
        wid = sid * NC + cid
        pltpu.sync_copy(zeros_hbm, zbuf)

        @pl.loop(0, np_pad // (ZCH * NS))
        def _(i):
            pltpu.sync_copy(zbuf, acc.at[pl.ds((i * NS + sid) * ZCH, ZCH)])

        plsc.subcore_barrier()
        base = wid * nblk * K

        def block(b):
            off = base + b * K
            pltpu.sync_copy(gidx_hbm.at[pl.ds(off, K)], gbuf)
            pltpu.sync_copy(didx_hbm.at[pl.ds(off, K)], dbuf)
            pltpu.async_copy(table_hbm.at[gbuf], rows, sem).wait()
            pltpu.sync_copy(rows, acc.at[dbuf], add=True)

        if dyn:
            pltpu.sync_copy(counts_hbm.at[wid], cbuf)
            cnt = lax.reduce_max(cbuf[pl.ds(0, 16)], axes=(0,))
            nb = (cnt + (K - 1)) // K
            lax.fori_loop(0, nb, lambda b, c: (block(b), c)[1], 0)
        else:
            pl.loop(0, nblk)(block)

        plsc.subcore_barrier()
        rps = np_pad // NS
        pltpu.sync_copy(acc.at[pl.ds(sid * rps, rps)],
                        out_hbm.at[cid, pl.ds(sid * rps, rps)])

    if dyn:
        return kern(table, gidx, didx, zeros_host, counts)
    return kern(table, gidx, didx, zeros_host)


def _aspect_gather(hmat, aidx):
    a = aidx.shape[0]
    h = hmat.shape[1]
    apt = a // NW

    @functools.partial(
        pl.kernel,
        out_type=jax.ShapeDtypeStruct((a, h), jnp.float32),
        mesh=_mesh(),
        scratch_types=[
            pltpu.VMEM((apt,), jnp.int32),
            pltpu.VMEM((apt, h), jnp.float32),
            pltpu.SemaphoreType.DMA,
        ],
    )
    def kern(h_hbm, aidx_hbm, out_hbm, ibuf, rows, sem):
        cid = lax.axis_index("c")
        sid = lax.axis_index("s")
        wid = sid * NC + cid
        pltpu.sync_copy(aidx_hbm.at[pl.ds(wid * apt, apt)], ibuf)
        pltpu.async_copy(h_hbm.at[ibuf], rows, sem).wait()
        pltpu.sync_copy(rows, out_hbm.at[pl.ds(wid * apt, apt)])

    return kern(hmat, aidx)


def _dinv_kernel(degs):
    np_pad = degs.shape[1]

    def body(d_ref, o_ref):
        deg = d_ref[0, :, 0:1] + d_ref[1, :, 0:1]
        o_ref[...] = lax.rsqrt(jnp.maximum(deg, 1.0))

    return pl.pallas_call(
        body,
        out_shape=jax.ShapeDtypeStruct((np_pad, 1), jnp.float32),
    )(degs)


def _typed_transform(x, w, dinv, bn=2048):
    np_pad, d = x.shape
    t, _, h = w.shape

    def body(x_ref, w_ref, dv_ref, o_ref):
        o_ref[0] = dv_ref[...] * jnp.dot(
            x_ref[...], w_ref[0], preferred_element_type=jnp.float32)

    return pl.pallas_call(
        body,
        grid=(t, np_pad // bn),
        in_specs=[
            pl.BlockSpec((bn, d), lambda ti, i: (i, 0)),
            pl.BlockSpec((1, d, h), lambda ti, i: (ti, 0, 0)),
            pl.BlockSpec((bn, 1), lambda ti, i: (i, 0)),
        ],
        out_specs=pl.BlockSpec((1, bn, h), lambda ti, i: (ti, i, 0)),
        out_shape=jax.ShapeDtypeStruct((t, np_pad, h), jnp.float32),
    )(x, w, dinv)


def _root_matmul(x, w_root, b, bn=2048):
    np_pad, d = x.shape
    h = w_root.shape[1]

    def body(x_ref, w_ref, b_ref, o_ref):
        o_ref[...] = jnp.dot(
            x_ref[...], w_ref[...], preferred_element_type=jnp.float32
        ) + b_ref[...]

    return pl.pallas_call(
        body,
        grid=(np_pad // bn,),
        in_specs=[
            pl.BlockSpec((bn, d), lambda i: (i, 0)),
            pl.BlockSpec((d, h), lambda i: (0, 0)),
            pl.BlockSpec((1, h), lambda i: (0, 0)),
        ],
        out_specs=pl.BlockSpec((bn, h), lambda i: (i, 0)),
        out_shape=jax.ShapeDtypeStruct((np_pad, h), jnp.float32),
    )(x, w_root, b)


def _combine(acc, dinv, root, relu, bn=2048):
    np_pad, h = root.shape

    def body(a_ref, dv_ref, r_ref, o_ref):
        s = (a_ref[0] + a_ref[1]) * dv_ref[...] + r_ref[...]
        if relu:
            s = jnp.maximum(s, 0.0)
        o_ref[...] = s

    return pl.pallas_call(
        body,
        grid=(np_pad // bn,),
        in_specs=[
            pl.BlockSpec((2, bn, h), lambda i: (0, i, 0)),
            pl.BlockSpec((bn, 1), lambda i: (i, 0)),
            pl.BlockSpec((bn, h), lambda i: (i, 0)),
        ],
        out_specs=pl.BlockSpec((bn, h), lambda i: (i, 0)),
        out_shape=jax.ShapeDtypeStruct((np_pad, h), jnp.float32),
    )(acc, dinv, root)


def _classifier(asp, wc, bc):
    a, h = asp.shape
    c = wc.shape[1]

    def body(x_ref, w_ref, b_ref, o_ref):
        o_ref[...] = jnp.dot(
            x_ref[...], w_ref[...], preferred_element_type=jnp.float32
        ) + b_ref[...]

    return pl.pallas_call(
        body,
        out_shape=jax.ShapeDtypeStruct((a, c), jnp.float32),
    )(asp, wc, bc.reshape(1, c))


def kernel(features, edge_index, aspect_indices, edge_types,
           W1, W1_root, b1, W2, W2_root, b2, Wc, bc):
    n, d = features.shape
    e = edge_index.shape[1]
    t = W1.shape[0]
    h = W1.shape[2]

    np_pad = -(-(n + 1) // (ZCH * NS)) * (ZCH * NS)
    e_pad = -(-e // (CH * K * NW)) * (CH * K * NW)

    src = edge_index[0].astype(jnp.int32)
    dst = edge_index[1].astype(jnp.int32)
    et = edge_types.astype(jnp.int32)
    pad = e_pad - e
    gidx = jnp.concatenate([et * np_pad + src,
                            jnp.zeros((pad,), jnp.int32)])
    didx = jnp.concatenate([dst, jnp.full((pad,), n, jnp.int32)])
    aidx = aspect_indices.astype(jnp.int32)

    x = jnp.pad(features, ((0, np_pad - n), (0, 0)))

    degs = _deg_pass(didx, np_pad)
    dinv = _dinv_kernel(degs)
    gidx2, didx2, counts2 = _filter_pass(gidx, didx, aidx, n, np_pad)

    xts1 = _typed_transform(x, W1, dinv).reshape(t * np_pad, h)
    acc1 = _edge_pass(xts1, gidx, didx, np_pad)
    root1 = _root_matmul(x, W1_root, b1.reshape(1, h))
    h1 = _combine(acc1, dinv, root1, relu=True)

    xts2 = _typed_transform(h1, W2, dinv).reshape(t * np_pad, h)
    acc2 = _edge_pass(xts2, gidx2, didx2, np_pad, counts2)
    root2 = _root_matmul(h1, W2_root, b2.reshape(1, h))
    h2 = _combine(acc2, dinv, root2, relu=False)

    asp = _aspect_gather(h2, aidx)
    return _classifier(asp, Wc, bc)


# speedup vs baseline: 2.6862x; 1.0933x over previous
import dataclasses
import functools

import jax
import jax.numpy as jnp
from jax import lax
from jax.experimental import pallas as pl
from jax.experimental.pallas import tpu as pltpu
from jax.experimental.pallas import tpu_sc as plsc

NC = 2
NS = 16
NW = NC * NS
K = 128
CH = 16
ZCH = 64


def _mesh():
    return plsc.VectorSubcoreMesh(core_axis_name="c", subcore_axis_name="s")


def _no_layout_cp():
    cp = pltpu.CompilerParams()
    if "needs_layout_passes" in pltpu.CompilerParams.__dataclass_fields__:
        cp = dataclasses.replace(cp, needs_layout_passes=False)
    return cp


def _deg_pass(didx, np_pad):
    nblk = didx.shape[0] // (K * NW)
    ones_host = jnp.ones((K, 128), jnp.float32)
    zeros_host = jnp.zeros((ZCH, 128), jnp.float32)

    @functools.partial(
        pl.kernel,
        out_type=jax.ShapeDtypeStruct((NC, np_pad, 128), jnp.float32),
        mesh=_mesh(),
        scratch_types=[
            pltpu.VMEM((K,), jnp.int32),
            pltpu.VMEM((K, 128), jnp.float32),
            pltpu.VMEM((ZCH, 128), jnp.float32),
            pltpu.VMEM_SHARED((np_pad, 128), jnp.float32),
        ],
    )
    def kern(didx_hbm, ones_hbm, zeros_hbm, out_hbm, dbuf, ones_v, zbuf, acc):
        cid = lax.axis_index("c")
        sid = lax.axis_index("s")
        wid = sid * NC + cid
        pltpu.sync_copy(ones_hbm, ones_v)
        pltpu.sync_copy(zeros_hbm, zbuf)

        @pl.loop(0, np_pad // (ZCH * NS))
        def _(i):
            pltpu.sync_copy(zbuf, acc.at[pl.ds((i * NS + sid) * ZCH, ZCH)])

        plsc.subcore_barrier()
        base = wid * nblk * K

        @pl.loop(0, nblk)
        def _(b):
            pltpu.sync_copy(didx_hbm.at[pl.ds(base + b * K, K)], dbuf)
            pltpu.sync_copy(ones_v, acc.at[dbuf], add=True)

        plsc.subcore_barrier()
        rps = np_pad // NS
        pltpu.sync_copy(acc.at[pl.ds(sid * rps, rps)],
                        out_hbm.at[cid, pl.ds(sid * rps, rps)])

    return kern(didx, ones_host, zeros_host)


def _filter_pass(gidx, didx, aidx, a_pad, np_pad):
    ep = gidx.shape[0]
    nblk = ep // (K * NW)
    ept = nblk * K
    a = aidx.shape[0]
    zflags_host = jnp.zeros((np_pad,), jnp.int32)
    junkd_host = jnp.full((ept,), a_pad - 1, jnp.int32)
    junkg_host = jnp.zeros((ept,), jnp.int32)

    @functools.partial(
        pl.kernel,
        out_type=(
            jax.ShapeDtypeStruct((ep,), jnp.int32),
            jax.ShapeDtypeStruct((ep,), jnp.int32),
            jax.ShapeDtypeStruct((NW, 128), jnp.int32),
            jax.ShapeDtypeStruct((a,), jnp.int32),
        ),
        mesh=_mesh(),
        scratch_types=[
            pltpu.VMEM((K,), jnp.int32),
            pltpu.VMEM((K,), jnp.int32),
            pltpu.VMEM((a,), jnp.int32),
            pltpu.VMEM((a,), jnp.int32),
            pltpu.VMEM((np_pad,), jnp.int32),
            pltpu.VMEM((ept,), jnp.int32),
            pltpu.VMEM((ept,), jnp.int32),
            pltpu.VMEM((128,), jnp.int32),
        ],
        compiler_params=_no_layout_cp(),
    )
    def kern(gidx_hbm, didx_hbm, aidx_hbm, zflags_hbm, junkd_hbm, junkg_hbm,
             gout_hbm, dout_hbm, counts_hbm, smap_hbm,
             gbuf, dbuf, abuf, sbuf, flag_v, gcomp, dcomp, cbuf):
        cid = lax.axis_index("c")
        sid = lax.axis_index("s")
        wid = sid * NC + cid
        pltpu.sync_copy(zflags_hbm, flag_v)
        pltpu.sync_copy(aidx_hbm, abuf)
        pltpu.sync_copy(junkd_hbm, dcomp)
        pltpu.sync_copy(junkg_hbm, gcomp)

        @pl.loop(0, a // 16)
        def _(c):
            slots = lax.iota(jnp.int32, 16) + (c * 16 + 1)
            plsc.store_scatter(flag_v, [abuf[pl.ds(c * 16, 16)]], slots)

        @pl.when(wid == 0)
        def _():
            @pl.loop(0, a // 16)
            def _(c):
                sv = plsc.load_gather(flag_v, [abuf[pl.ds(c * 16, 16)]]) - 1
                sbuf[pl.ds(c * 16, 16)] = sv

            pltpu.sync_copy(sbuf, smap_hbm)

        base = wid * ept

        def blk_body(b, cnt):
            pltpu.sync_copy(gidx_hbm.at[pl.ds(base + b * K, K)], gbuf)
            pltpu.sync_copy(didx_hbm.at[pl.ds(base + b * K, K)], dbuf)

            def chunk(c, cnt):
                dv = dbuf[pl.ds(c * 16, 16)]
                gv = gbuf[pl.ds(c * 16, 16)]
                fl = plsc.load_gather(flag_v, [dv])
                mask = fl > 0
                plsc.store_compressed(dcomp.at[pl.ds(cnt, 16)], fl - 1,
                                      mask=mask)
                plsc.store_compressed(gcomp.at[pl.ds(cnt, 16)], gv, mask=mask)
                pc = plsc.all_reduce_population_count(mask)
                return cnt + lax.reduce_max(pc, axes=(0,))

            return lax.fori_loop(0, K // 16, chunk, cnt)

        cnt = lax.fori_loop(0, nblk, blk_body, jnp.int32(0))
        pltpu.sync_copy(dcomp, dout_hbm.at[pl.ds(base, ept)])
        pltpu.sync_copy(gcomp, gout_hbm.at[pl.ds(base, ept)])
        cvec = jnp.full((16,), cnt, jnp.int32)

        @pl.loop(0, 8)
        def _(j):
            cbuf[pl.ds(j * 16, 16)] = cvec

        pltpu.sync_copy(cbuf, counts_hbm.at[wid])

    return kern(gidx, didx, aidx, zflags_host, junkd_host, junkg_host)


def _edge_pass(table, gidx, didx, n_acc, counts=None):
    h = table.shape[1]
    nblk = gidx.shape[0] // (K * NW)
    zeros_host = jnp.zeros((ZCH, h), jnp.float32)
    dyn = counts is not None
    rps = n_acc // NS

    scratch = [
        pltpu.VMEM((K,), jnp.int32),
        pltpu.VMEM((K,), jnp.int32),
        pltpu.VMEM((K, h), jnp.float32),
        pltpu.VMEM((ZCH, h), jnp.float32),
        pltpu.VMEM_SHARED((n_acc, h), jnp.float32),
        pltpu.SemaphoreType.DMA,
    ]
    if dyn:
        scratch.append(pltpu.VMEM((128,), jnp.int32))

    @functools.partial(
        pl.kernel,
        out_type=jax.ShapeDtypeStruct((NC, n_acc, h), jnp.float32),
        mesh=_mesh(),
        scratch_types=scratch,
        compiler_params=_no_layout_cp() if dyn else None,
    )
    def kern(table_hbm, gidx_hbm, didx_hbm, zeros_hbm, *rest):
        if dyn:
            counts_hbm, out_hbm, gbuf, dbuf, rows, zbuf, acc, sem, cbuf = rest
        else:
            out_hbm, gbuf, dbuf, rows, zbuf, acc, sem = rest
        cid = lax.axis_index("c")
        sid = lax.axis_index("s")
        wid = sid * NC + cid
        pltpu.sync_copy(zeros_hbm, zbuf)

        for j in range(rps // ZCH):
            pltpu.sync_copy(zbuf, acc.at[pl.ds(sid * rps + j * ZCH, ZCH)])
        if rps % ZCH:
            r0 = (rps // ZCH) * ZCH
            pltpu.sync_copy(zbuf.at[pl.ds(0, rps % ZCH)],
                            acc.at[pl.ds(sid * rps + r0, rps % ZCH)])

        plsc.subcore_barrier()
        base = wid * nblk * K

        def block(b):
            off = base + b * K
            pltpu.sync_copy(gidx_hbm.at[pl.ds(off, K)], gbuf)
            pltpu.sync_copy(didx_hbm.at[pl.ds(off, K)], dbuf)
            pltpu.async_copy(table_hbm.at[gbuf], rows, sem).wait()
            pltpu.sync_copy(rows, acc.at[dbuf], add=True)

        if dyn:
            pltpu.sync_copy(counts_hbm.at[wid], cbuf)
            cnt = lax.reduce_max(cbuf[pl.ds(0, 16)], axes=(0,))
            nb = (cnt + (K - 1)) // K
            lax.fori_loop(0, nb, lambda b, c: (block(b), c)[1], 0)
        else:
            pl.loop(0, nblk)(block)

        plsc.subcore_barrier()
        pltpu.sync_copy(acc.at[pl.ds(sid * rps, rps)],
                        out_hbm.at[cid, pl.ds(sid * rps, rps)])

    if dyn:
        return kern(table, gidx, didx, zeros_host, counts)
    return kern(table, gidx, didx, zeros_host)


def _aspect_gather(hmat, aidx):
    a = aidx.shape[0]
    h = hmat.shape[1]
    apt = a // NW

    @functools.partial(
        pl.kernel,
        out_type=jax.ShapeDtypeStruct((a, h), jnp.float32),
        mesh=_mesh(),
        scratch_types=[
            pltpu.VMEM((apt,), jnp.int32),
            pltpu.VMEM((apt, h), jnp.float32),
            pltpu.SemaphoreType.DMA,
        ],
    )
    def kern(h_hbm, aidx_hbm, out_hbm, ibuf, rows, sem):
        cid = lax.axis_index("c")
        sid = lax.axis_index("s")
        wid = sid * NC + cid
        pltpu.sync_copy(aidx_hbm.at[pl.ds(wid * apt, apt)], ibuf)
        pltpu.async_copy(h_hbm.at[ibuf], rows, sem).wait()
        pltpu.sync_copy(rows, out_hbm.at[pl.ds(wid * apt, apt)])

    return kern(hmat, aidx)


def _dinv_kernel(degs):
    np_pad = degs.shape[1]

    def body(d_ref, o_ref):
        deg = d_ref[0, :, 0:1] + d_ref[1, :, 0:1]
        o_ref[...] = lax.rsqrt(jnp.maximum(deg, 1.0))

    return pl.pallas_call(
        body,
        out_shape=jax.ShapeDtypeStruct((np_pad, 1), jnp.float32),
    )(degs)


def _typed_transform(x, w, dinv, bn=2048):
    np_pad, d = x.shape
    t, _, h = w.shape

    def body(x_ref, w_ref, dv_ref, o_ref):
        o_ref[0] = dv_ref[...] * jnp.dot(
            x_ref[...], w_ref[0], preferred_element_type=jnp.float32)

    return pl.pallas_call(
        body,
        grid=(t, np_pad // bn),
        in_specs=[
            pl.BlockSpec((bn, d), lambda ti, i: (i, 0)),
            pl.BlockSpec((1, d, h), lambda ti, i: (ti, 0, 0)),
            pl.BlockSpec((bn, 1), lambda ti, i: (i, 0)),
        ],
        out_specs=pl.BlockSpec((1, bn, h), lambda ti, i: (ti, i, 0)),
        out_shape=jax.ShapeDtypeStruct((t, np_pad, h), jnp.float32),
    )(x, w, dinv)


def _root_matmul(x, w_root, b, bn=2048):
    np_pad, d = x.shape
    h = w_root.shape[1]

    def body(x_ref, w_ref, b_ref, o_ref):
        o_ref[...] = jnp.dot(
            x_ref[...], w_ref[...], preferred_element_type=jnp.float32
        ) + b_ref[...]

    return pl.pallas_call(
        body,
        grid=(np_pad // bn,),
        in_specs=[
            pl.BlockSpec((bn, d), lambda i: (i, 0)),
            pl.BlockSpec((d, h), lambda i: (0, 0)),
            pl.BlockSpec((1, h), lambda i: (0, 0)),
        ],
        out_specs=pl.BlockSpec((bn, h), lambda i: (i, 0)),
        out_shape=jax.ShapeDtypeStruct((np_pad, h), jnp.float32),
    )(x, w_root, b)


def _combine(acc, dinv, root, relu, bn=2048):
    np_pad, h = root.shape

    def body(a_ref, dv_ref, r_ref, o_ref):
        s = (a_ref[0] + a_ref[1]) * dv_ref[...] + r_ref[...]
        if relu:
            s = jnp.maximum(s, 0.0)
        o_ref[...] = s

    return pl.pallas_call(
        body,
        grid=(np_pad // bn,),
        in_specs=[
            pl.BlockSpec((2, bn, h), lambda i: (0, i, 0)),
            pl.BlockSpec((bn, 1), lambda i: (i, 0)),
            pl.BlockSpec((bn, h), lambda i: (i, 0)),
        ],
        out_specs=pl.BlockSpec((bn, h), lambda i: (i, 0)),
        out_shape=jax.ShapeDtypeStruct((np_pad, h), jnp.float32),
    )(acc, dinv, root)


def _aspect_combine(acc, dasp, asp_h1, w_root, b):
    a, h = asp_h1.shape

    def body(a_ref, d_ref, x_ref, w_ref, b_ref, o_ref):
        deg = d_ref[0:a, 0:1] + d_ref[a:2 * a, 0:1]
        dinv = lax.rsqrt(jnp.maximum(deg, 1.0))
        o_ref[...] = (a_ref[0] + a_ref[1]) * dinv + jnp.dot(
            x_ref[...], w_ref[...], preferred_element_type=jnp.float32
        ) + b_ref[...]

    return pl.pallas_call(
        body,
        out_shape=jax.ShapeDtypeStruct((a, h), jnp.float32),
    )(acc, dasp, asp_h1, w_root, b.reshape(1, h))


def _classifier(asp, wc, bc):
    a, h = asp.shape
    c = wc.shape[1]

    def body(x_ref, w_ref, b_ref, o_ref):
        o_ref[...] = jnp.dot(
            x_ref[...], w_ref[...], preferred_element_type=jnp.float32
        ) + b_ref[...]

    return pl.pallas_call(
        body,
        out_shape=jax.ShapeDtypeStruct((a, c), jnp.float32),
    )(asp, wc, bc.reshape(1, c))


def kernel(features, edge_index, aspect_indices, edge_types,
           W1, W1_root, b1, W2, W2_root, b2, Wc, bc):
    n, d = features.shape
    e = edge_index.shape[1]
    t = W1.shape[0]
    h = W1.shape[2]

    np_pad = -(-(n + 1) // (ZCH * NS)) * (ZCH * NS)
    e_pad = -(-e // (CH * K * NW)) * (CH * K * NW)

    src = edge_index[0].astype(jnp.int32)
    dst = edge_index[1].astype(jnp.int32)
    et = edge_types.astype(jnp.int32)
    pad = e_pad - e
    gidx = jnp.concatenate([et * np_pad + src,
                            jnp.zeros((pad,), jnp.int32)])
    didx = jnp.concatenate([dst, jnp.full((pad,), n, jnp.int32)])
    aidx = aspect_indices.astype(jnp.int32)

    x = jnp.pad(features, ((0, np_pad - n), (0, 0)))

    a = aidx.shape[0]
    a_pad = -(-(a + 1) // 128) * 128

    degs = _deg_pass(didx, np_pad)
    dinv = _dinv_kernel(degs)
    gidx2, didx2, counts2, smap = _filter_pass(gidx, didx, aidx, a_pad,
                                               np_pad)

    xts1 = _typed_transform(x, W1, dinv).reshape(t * np_pad, h)
    acc1 = _edge_pass(xts1, gidx, didx, np_pad)
    root1 = _root_matmul(x, W1_root, b1.reshape(1, h))
    h1 = _combine(acc1, dinv, root1, relu=True)

    xts2 = _typed_transform(h1, W2, dinv).reshape(t * np_pad, h)
    acc2 = _edge_pass(xts2, gidx2, didx2, a_pad, counts2)
    asp_h1 = _aspect_gather(h1, aidx)
    aidx2 = jnp.concatenate([aidx, aidx + np_pad])
    dasp = _aspect_gather(degs.reshape(NC * np_pad, 128), aidx2)
    h2a = _aspect_combine(acc2[:, :a], dasp, asp_h1, W2_root, b2)
    h2f = _aspect_gather(h2a, smap)
    return _classifier(h2f, Wc, bc)
